# baseline (speedup 1.0000x reference)
import functools
import jax
import jax.numpy as jnp
from jax import lax
from jax.experimental import pallas as pl
from jax.experimental.pallas import tpu as pltpu
from jax.experimental.pallas import tpu_sc as plsc

PAD = 1
L = 16
CH = 32


def _make_sc(bsz, seq_len, d, num_rows):
    info = plsc.get_sparse_core_info()
    nw = info.num_cores * info.num_subcores
    total = bsz * seq_len
    rows_w = total // nw
    nch = rows_w // CH
    wpb = nw // bsz
    assert seq_len % rows_w == 0 and rows_w % CH == 0 and total % nw == 0
    mesh = plsc.VectorSubcoreMesh(core_axis_name="c", subcore_axis_name="s")
    nc = info.num_cores

    @functools.partial(
        pl.kernel,
        mesh=mesh,
        out_type=jax.ShapeDtypeStruct((total, d), jnp.float32),
        scratch_types=[
            pltpu.VMEM((rows_w,), jnp.int32),
            pltpu.VMEM((nch, CH), jnp.int32),
            pltpu.VMEM((CH, d), jnp.float32),
            pltpu.VMEM((CH, d), jnp.float32),
            pltpu.SemaphoreType.DMA,
            pltpu.SemaphoreType.DMA,
            pltpu.SemaphoreType.DMA,
            pltpu.SemaphoreType.DMA,
        ],
    )
    def k(inp_hbm, table_hbm, out_hbm, tok_v, idx_v, buf0, buf1, g0, g1, s0, s1):
        wid = lax.axis_index("s") * nc + lax.axis_index("c")
        base = wid * rows_w
        jbase = lax.rem(wid, wpb) * rows_w + 2

        pltpu.sync_copy(inp_hbm.at[pl.ds(base, rows_w)], tok_v)

        lane = jnp.arange(L, dtype=jnp.int32)
        for ch in range(nch):
            for v in range(CH // L):
                o = ch * CH + v * L
                tok = tok_v[pl.ds(o, L)]
                pos = lane + (jbase + o)
                idx_v[ch, pl.ds(v * L, L)] = jnp.where(tok != PAD, pos, PAD)

        bufs = (buf0, buf1)
        gsems = (g0, g1)
        ssems = (s0, s1)

        pltpu.async_copy(table_hbm.at[idx_v.at[0]], bufs[0], gsems[0])
        for ch in range(nch):
            p = ch % 2
            q = 1 - p
            pltpu.make_async_copy(table_hbm.at[idx_v.at[ch]], bufs[p], gsems[p]).wait()
            if ch + 1 < nch:
                if ch >= 1:
                    pltpu.make_async_copy(
                        bufs[q], out_hbm.at[pl.ds(base + (ch - 1) * CH, CH)], ssems[q]
                    ).wait()
                pltpu.async_copy(table_hbm.at[idx_v.at[ch + 1]], bufs[q], gsems[q])
            pltpu.async_copy(bufs[p], out_hbm.at[pl.ds(base + ch * CH, CH)], ssems[p])
        for ch in (nch - 2, nch - 1):
            p = ch % 2
            pltpu.make_async_copy(
                bufs[p], out_hbm.at[pl.ds(base + ch * CH, CH)], ssems[p]
            ).wait()

    return k


def kernel(input, weights):
    bsz, seq_len = input.shape
    d = weights.shape[1]
    k = _make_sc(bsz, seq_len, d, weights.shape[0])
    out = k(input.reshape(-1), weights)
    return out.reshape(bsz, seq_len, d)
